# baseline (speedup 1.0000x reference)
import functools

import jax
import jax.numpy as jnp
from jax import lax
from jax.experimental import pallas as pl
from jax.experimental.pallas import tpu as pltpu
from jax.experimental.pallas import tpu_sc as plsc

B = 16384
H = 64
AL = 128
ML = 100
SL = 128
OUT_COLS = 576

NW = 32
BPW = B // NW
ICH = 128
NCH = BPW // ICH

_EPS = 1e-5


def _sc_gather(uid, age, singer, genre, mid,
               user_t, age_t, singer_t, genre_t, music_t):
    mesh = plsc.VectorSubcoreMesh(core_axis_name="c", subcore_axis_name="s",
                                  num_cores=2, num_subcores=16)

    @functools.partial(
        pl.kernel,
        mesh=mesh,
        out_type=jax.ShapeDtypeStruct((B, OUT_COLS), jnp.float32),
        compiler_params=pltpu.CompilerParams(use_tc_tiling_on_sc=False),
        scratch_types=[
            pltpu.VMEM((NCH, ICH), jnp.int32),
            pltpu.VMEM((BPW, H), jnp.float32),
            pltpu.SemaphoreType.DMA,
        ],
    )
    def body(uid_h, age_h, sing_h, genr_h, mid_h,
             ut_h, at_h, st_h, gr_h, mt_h, out_h,
             idx_v, rows_v, sem):
        wid = lax.axis_index("s") * 2 + lax.axis_index("c")
        base = wid * BPW
        fields = ((uid_h, ut_h, 0), (age_h, at_h, 64),
                  (sing_h, st_h, 384), (genr_h, gr_h, 448), (mid_h, mt_h, 512))
        for idx_h, tab_h, col in fields:
            pltpu.sync_copy(idx_h.at[wid], idx_v)
            cps = [
                pltpu.async_copy(tab_h.at[idx_v.at[j]],
                                 rows_v.at[pl.ds(j * ICH, ICH)], sem)
                for j in range(NCH)
            ]
            for cp in cps:
                cp.wait()
            pltpu.sync_copy(rows_v, out_h.at[pl.ds(base, BPW), pl.ds(col, H)])

    r = lambda x: jnp.reshape(x.astype(jnp.int32), (NW, NCH, ICH))
    return body(r(uid), r(age), r(singer), r(genre), r(mid),
                user_t, age_t, singer_t, genre_t, music_t)


_NB = 32
_BB = B // _NB


def _stats(art, mom, feat):
    def body(a_ref, m_ref, f_ref, sa, qa, sm, qm, sf, qf):
        @pl.when(pl.program_id(0) == 0)
        def _():
            for r in (sa, qa, sm, qm, sf, qf):
                r[...] = jnp.zeros_like(r)

        for x_ref, s_ref, q_ref in ((a_ref, sa, qa), (m_ref, sm, qm),
                                    (f_ref, sf, qf)):
            x = x_ref[...]
            s_ref[...] += jnp.sum(x, axis=0, keepdims=True)
            q_ref[...] += jnp.sum(x * x, axis=0, keepdims=True)

    stat_spec = lambda k: pl.BlockSpec((1, k), lambda i: (0, 0))
    return pl.pallas_call(
        body,
        grid=(_NB,),
        in_specs=[
            pl.BlockSpec((_BB, AL), lambda i: (i, 0)),
            pl.BlockSpec((_BB, ML), lambda i: (i, 0)),
            pl.BlockSpec((_BB, SL), lambda i: (i, 0)),
        ],
        out_specs=[stat_spec(AL), stat_spec(AL), stat_spec(ML),
                   stat_spec(ML), stat_spec(SL), stat_spec(SL)],
        out_shape=[jax.ShapeDtypeStruct((1, k), jnp.float32)
                   for k in (AL, AL, ML, ML, SL, SL)],
    )(art, mom, feat)


def _dense(out0, gen_f, gender_t, art, mom, feat,
           w_uf, b_uf, w_ml, b_ml, w_sf, b_sf,
           g_art, be_art, g_mom, be_mom, g_feat, be_feat,
           sa, qa, sm, qm, sf, qf):
    def body(o_any, u_ref, gt_ref, a_ref, m_ref, f_ref,
             wa, ba, wm, bm, wf, bf,
             ga, bea, gm, bem, gf, bef,
             sa_r, qa_r, sm_r, qm_r, sf_r, qf_r, out_ref, y_v, sem):
        del o_any
        g0 = gt_ref[0:1, :]
        y_v[:, 0:H] = g0 + u_ref[...] * (gt_ref[1:2, :] - g0)
        for x_ref, w_ref, b_ref, g_ref, be_ref, s_ref, q_ref, off in (
                (a_ref, wa, ba, ga, bea, sa_r, qa_r, 64),
                (m_ref, wm, bm, gm, bem, sm_r, qm_r, 128),
                (f_ref, wf, bf, gf, bef, sf_r, qf_r, 192)):
            mu = s_ref[...] * (1.0 / B)
            var = q_ref[...] * (1.0 / B) - mu * mu
            sc = g_ref[...] / jnp.sqrt(var + _EPS)
            sh = be_ref[...] - mu * sc
            xn = x_ref[...] * sc + sh
            y = jnp.dot(xn, w_ref[...], preferred_element_type=jnp.float32)
            y_v[:, off:off + H] = y + b_ref[...]
        i = pl.program_id(0)
        cp = pltpu.make_async_copy(
            y_v, out_ref.at[pl.ds(i * _BB, _BB), pl.ds(128, 256)], sem)
        cp.start()
        cp.wait()

    full = lambda r, c: pl.BlockSpec((r, c), lambda i: (0, 0))
    return pl.pallas_call(
        body,
        grid=(_NB,),
        in_specs=[
            pl.BlockSpec(memory_space=pl.ANY),
            pl.BlockSpec((_BB, 1), lambda i: (i, 0)),
            full(2, H),
            pl.BlockSpec((_BB, AL), lambda i: (i, 0)),
            pl.BlockSpec((_BB, ML), lambda i: (i, 0)),
            pl.BlockSpec((_BB, SL), lambda i: (i, 0)),
            full(AL, H), full(1, H), full(ML, H), full(1, H),
            full(SL, H), full(1, H),
            full(1, AL), full(1, AL), full(1, ML), full(1, ML),
            full(1, SL), full(1, SL),
            full(1, AL), full(1, AL), full(1, ML), full(1, ML),
            full(1, SL), full(1, SL),
        ],
        out_specs=pl.BlockSpec(memory_space=pl.ANY),
        out_shape=jax.ShapeDtypeStruct((B, OUT_COLS), jnp.float32),
        scratch_shapes=[pltpu.VMEM((_BB, 256), jnp.float32),
                        pltpu.SemaphoreType.DMA],
        input_output_aliases={0: 0},
    )(out0, gen_f, gender_t, art, mom, feat,
      w_uf, b_uf.reshape(1, H), w_ml, b_ml.reshape(1, H),
      w_sf, b_sf.reshape(1, H),
      g_art.reshape(1, AL), be_art.reshape(1, AL),
      g_mom.reshape(1, ML), be_mom.reshape(1, ML),
      g_feat.reshape(1, SL), be_feat.reshape(1, SL),
      sa, qa, sm, qm, sf, qf)


def kernel(user_id, user_age, user_gender, user_articles, user_moments,
           music_id, music_singer, music_genre, music_features,
           UserEmb, AgeEmb, GenderEmb, SingerEmb, GenreEmb, MusicEmb,
           W_uf, b_uf, W_ml, b_ml, W_sf, b_sf,
           g_art, beta_art, g_mom, beta_mom, g_feat, beta_feat):
    out = _sc_gather(user_id, user_age, music_singer, music_genre, music_id,
                     UserEmb, AgeEmb, SingerEmb, GenreEmb, MusicEmb)
    sa, qa, sm, qm, sf, qf = _stats(user_articles, user_moments,
                                    music_features)
    gen_f = user_gender.astype(jnp.float32).reshape(B, 1)
    return _dense(out, gen_f, GenderEmb,
                  user_articles, user_moments, music_features,
                  W_uf, b_uf, W_ml, b_ml, W_sf, b_sf,
                  g_art, beta_art, g_mom, beta_mom, g_feat, beta_feat,
                  sa, qa, sm, qm, sf, qf)


# speedup vs baseline: 1.0100x; 1.0100x over previous
import functools

import jax
import jax.numpy as jnp
from jax import lax
from jax.experimental import pallas as pl
from jax.experimental.pallas import tpu as pltpu
from jax.experimental.pallas import tpu_sc as plsc

B = 16384
H = 64
AL = 128
ML = 100
SL = 128
OUT_COLS = 576

NW = 32
BPW = B // NW
ICH = 128
NCH = BPW // ICH

_EPS = 1e-5


def _sc_gather(uid, age, singer, genre, mid,
               user_t, age_t, singer_t, genre_t, music_t):
    mesh = plsc.VectorSubcoreMesh(core_axis_name="c", subcore_axis_name="s",
                                  num_cores=2, num_subcores=16)
    out64 = jax.ShapeDtypeStruct((B, H), jnp.float32)

    @functools.partial(
        pl.kernel,
        mesh=mesh,
        out_type=(out64, out64, out64, out64, out64),
        compiler_params=pltpu.CompilerParams(use_tc_tiling_on_sc=False),
        scratch_types=[
            pltpu.VMEM((NCH, ICH), jnp.int32),
            pltpu.VMEM((NCH, ICH), jnp.int32),
            pltpu.VMEM((BPW, H), jnp.float32),
            pltpu.VMEM((BPW, H), jnp.float32),
            pltpu.SemaphoreType.DMA,
            pltpu.SemaphoreType.DMA,
            pltpu.SemaphoreType.DMA,
            pltpu.SemaphoreType.DMA,
        ],
    )
    def body(uid_h, age_h, sing_h, genr_h, mid_h,
             ut_h, at_h, st_h, gt_h, mt_h,
             uo_h, ao_h, so_h, go_h, mo_h,
             idx0, idx1, buf0, buf1, g0, g1, w0, w1):
        wid = lax.axis_index("s") * 2 + lax.axis_index("c")
        base = wid * BPW
        fields = ((uid_h, ut_h, uo_h), (age_h, at_h, ao_h),
                  (sing_h, st_h, so_h), (genr_h, gt_h, go_h),
                  (mid_h, mt_h, mo_h))
        idxs = (idx0, idx1)
        bufs = (buf0, buf1)
        gsems = (g0, g1)
        wsems = (w0, w1)
        pend_g = [None, None]
        pend_w = [None, None]
        for f, (idx_h, tab_h, out_h) in enumerate(fields):
            p = f % 2
            if pend_w[p] is not None:
                pend_w[p].wait()
                pend_w[p] = None
            pltpu.sync_copy(idx_h.at[wid], idxs[p])
            pend_g[p] = [
                pltpu.async_copy(tab_h.at[idxs[p].at[j]],
                                 bufs[p].at[pl.ds(j * ICH, ICH)], gsems[p])
                for j in range(NCH)
            ]
            q = 1 - p
            if pend_g[q] is not None:
                for cp in pend_g[q]:
                    cp.wait()
                pend_g[q] = None
                prev_out = fields[f - 1][2]
                pend_w[q] = pltpu.async_copy(
                    bufs[q], prev_out.at[pl.ds(base, BPW)], wsems[q])
        p = (len(fields) - 1) % 2
        for cp in pend_g[p]:
            cp.wait()
        pend_w[p] = pltpu.async_copy(
            bufs[p], fields[-1][2].at[pl.ds(base, BPW)], wsems[p])
        for q in (0, 1):
            if pend_w[q] is not None:
                pend_w[q].wait()

    r = lambda x: jnp.reshape(x.astype(jnp.int32), (NW, NCH, ICH))
    return body(r(uid), r(age), r(singer), r(genre), r(mid),
                user_t, age_t, singer_t, genre_t, music_t)


_NB = 32
_BB = B // _NB


def _stats(art, mom, feat):
    def body(a_ref, m_ref, f_ref, sa, qa, sm, qm, sf, qf):
        @pl.when(pl.program_id(0) == 0)
        def _():
            for r in (sa, qa, sm, qm, sf, qf):
                r[...] = jnp.zeros_like(r)

        for x_ref, s_ref, q_ref in ((a_ref, sa, qa), (m_ref, sm, qm),
                                    (f_ref, sf, qf)):
            x = x_ref[...]
            s_ref[...] += jnp.sum(x, axis=0, keepdims=True)
            q_ref[...] += jnp.sum(x * x, axis=0, keepdims=True)

    stat_spec = lambda k: pl.BlockSpec((1, k), lambda i: (0, 0))
    return pl.pallas_call(
        body,
        grid=(_NB,),
        in_specs=[
            pl.BlockSpec((_BB, AL), lambda i: (i, 0)),
            pl.BlockSpec((_BB, ML), lambda i: (i, 0)),
            pl.BlockSpec((_BB, SL), lambda i: (i, 0)),
        ],
        out_specs=[stat_spec(AL), stat_spec(AL), stat_spec(ML),
                   stat_spec(ML), stat_spec(SL), stat_spec(SL)],
        out_shape=[jax.ShapeDtypeStruct((1, k), jnp.float32)
                   for k in (AL, AL, ML, ML, SL, SL)],
    )(art, mom, feat)


def _dense(uemb, aemb, semb, gemb, memb, gen_f, gender_t, art, mom, feat,
           w_uf, b_uf, w_ml, b_ml, w_sf, b_sf,
           g_art, be_art, g_mom, be_mom, g_feat, be_feat,
           sa, qa, sm, qm, sf, qf):
    def body(ue_ref, ae_ref, se_ref, ge_ref, me_ref,
             u_ref, gt_ref, a_ref, m_ref, f_ref,
             wa, ba, wm, bm, wf, bf,
             ga, bea, gm, bem, gf, bef,
             sa_r, qa_r, sm_r, qm_r, sf_r, qf_r, out_ref):
        out_ref[:, 0:H] = ue_ref[...]
        out_ref[:, H:2 * H] = ae_ref[...]
        g0 = gt_ref[0:1, :]
        out_ref[:, 2 * H:3 * H] = g0 + u_ref[...] * (gt_ref[1:2, :] - g0)
        for x_ref, w_ref, b_ref, g_ref, be_ref, s_ref, q_ref, off in (
                (a_ref, wa, ba, ga, bea, sa_r, qa_r, 3 * H),
                (m_ref, wm, bm, gm, bem, sm_r, qm_r, 4 * H),
                (f_ref, wf, bf, gf, bef, sf_r, qf_r, 5 * H)):
            mu = s_ref[...] * (1.0 / B)
            var = q_ref[...] * (1.0 / B) - mu * mu
            sc = g_ref[...] / jnp.sqrt(var + _EPS)
            sh = be_ref[...] - mu * sc
            xn = x_ref[...] * sc + sh
            y = jnp.dot(xn, w_ref[...], preferred_element_type=jnp.float32)
            out_ref[:, off:off + H] = y + b_ref[...]
        out_ref[:, 6 * H:7 * H] = se_ref[...]
        out_ref[:, 7 * H:8 * H] = ge_ref[...]
        out_ref[:, 8 * H:9 * H] = me_ref[...]

    full = lambda r, c: pl.BlockSpec((r, c), lambda i: (0, 0))
    row64 = pl.BlockSpec((_BB, H), lambda i: (i, 0))
    return pl.pallas_call(
        body,
        grid=(_NB,),
        in_specs=[
            row64, row64, row64, row64, row64,
            pl.BlockSpec((_BB, 1), lambda i: (i, 0)),
            full(2, H),
            pl.BlockSpec((_BB, AL), lambda i: (i, 0)),
            pl.BlockSpec((_BB, ML), lambda i: (i, 0)),
            pl.BlockSpec((_BB, SL), lambda i: (i, 0)),
            full(AL, H), full(1, H), full(ML, H), full(1, H),
            full(SL, H), full(1, H),
            full(1, AL), full(1, AL), full(1, ML), full(1, ML),
            full(1, SL), full(1, SL),
            full(1, AL), full(1, AL), full(1, ML), full(1, ML),
            full(1, SL), full(1, SL),
        ],
        out_specs=pl.BlockSpec((_BB, OUT_COLS), lambda i: (i, 0)),
        out_shape=jax.ShapeDtypeStruct((B, OUT_COLS), jnp.float32),
    )(uemb, aemb, semb, gemb, memb, gen_f, gender_t, art, mom, feat,
      w_uf, b_uf.reshape(1, H), w_ml, b_ml.reshape(1, H),
      w_sf, b_sf.reshape(1, H),
      g_art.reshape(1, AL), be_art.reshape(1, AL),
      g_mom.reshape(1, ML), be_mom.reshape(1, ML),
      g_feat.reshape(1, SL), be_feat.reshape(1, SL),
      sa, qa, sm, qm, sf, qf)


def kernel(user_id, user_age, user_gender, user_articles, user_moments,
           music_id, music_singer, music_genre, music_features,
           UserEmb, AgeEmb, GenderEmb, SingerEmb, GenreEmb, MusicEmb,
           W_uf, b_uf, W_ml, b_ml, W_sf, b_sf,
           g_art, beta_art, g_mom, beta_mom, g_feat, beta_feat):
    uemb, aemb, semb, gemb, memb = _sc_gather(
        user_id, user_age, music_singer, music_genre, music_id,
        UserEmb, AgeEmb, SingerEmb, GenreEmb, MusicEmb)
    sa, qa, sm, qm, sf, qf = _stats(user_articles, user_moments,
                                    music_features)
    gen_f = user_gender.astype(jnp.float32).reshape(B, 1)
    return _dense(uemb, aemb, semb, gemb, memb, gen_f, GenderEmb,
                  user_articles, user_moments, music_features,
                  W_uf, b_uf, W_ml, b_ml, W_sf, b_sf,
                  g_art, beta_art, g_mom, beta_mom, g_feat, beta_feat,
                  sa, qa, sm, qm, sf, qf)


# speedup vs baseline: 1.5856x; 1.5700x over previous
import functools

import jax
import jax.numpy as jnp
from jax import lax
from jax.experimental import pallas as pl
from jax.experimental.pallas import tpu as pltpu
from jax.experimental.pallas import tpu_sc as plsc

B = 16384
H = 64
AL = 128
ML = 100
SL = 128
OUT_COLS = 576

NW = 32
BPW = B // NW
ICH = 128
NCH = BPW // ICH

_EPS = 1e-5


def _sc_gather(uid, mid, user_t, music_t):
    mesh = plsc.VectorSubcoreMesh(core_axis_name="c", subcore_axis_name="s",
                                  num_cores=2, num_subcores=16)
    out64 = jax.ShapeDtypeStruct((B, H), jnp.float32)

    @functools.partial(
        pl.kernel,
        mesh=mesh,
        out_type=(out64, out64),
        compiler_params=pltpu.CompilerParams(use_tc_tiling_on_sc=False),
        scratch_types=[
            pltpu.VMEM((NCH, ICH), jnp.int32),
            pltpu.VMEM((NCH, ICH), jnp.int32),
            pltpu.VMEM((BPW, H), jnp.float32),
            pltpu.VMEM((BPW, H), jnp.float32),
            pltpu.SemaphoreType.DMA,
            pltpu.SemaphoreType.DMA,
            pltpu.SemaphoreType.DMA,
            pltpu.SemaphoreType.DMA,
        ],
    )
    def body(uid_h, mid_h, ut_h, mt_h, uo_h, mo_h,
             idx0, idx1, buf0, buf1, g0, g1, w0, w1):
        wid = lax.axis_index("s") * 2 + lax.axis_index("c")
        base = wid * BPW
        pltpu.sync_copy(uid_h.at[wid], idx0)
        gu = [pltpu.async_copy(ut_h.at[idx0.at[j]],
                               buf0.at[pl.ds(j * ICH, ICH)], g0)
              for j in range(NCH)]
        pltpu.sync_copy(mid_h.at[wid], idx1)
        gm = [pltpu.async_copy(mt_h.at[idx1.at[j]],
                               buf1.at[pl.ds(j * ICH, ICH)], g1)
              for j in range(NCH)]
        for cp in gu:
            cp.wait()
        wu = pltpu.async_copy(buf0, uo_h.at[pl.ds(base, BPW)], w0)
        for cp in gm:
            cp.wait()
        wm = pltpu.async_copy(buf1, mo_h.at[pl.ds(base, BPW)], w1)
        wu.wait()
        wm.wait()

    r = lambda x: jnp.reshape(x.astype(jnp.int32), (NW, NCH, ICH))
    return body(r(uid), r(mid), user_t, music_t)


_NB = 16
_BB = B // _NB


def _stats(art, mom, feat):
    def body(a_ref, m_ref, f_ref, sa, qa, sm, qm, sf, qf):
        @pl.when(pl.program_id(0) == 0)
        def _():
            for r in (sa, qa, sm, qm, sf, qf):
                r[...] = jnp.zeros_like(r)

        for x_ref, s_ref, q_ref in ((a_ref, sa, qa), (m_ref, sm, qm),
                                    (f_ref, sf, qf)):
            x = x_ref[...]
            s_ref[...] += jnp.sum(x, axis=0, keepdims=True)
            q_ref[...] += jnp.sum(x * x, axis=0, keepdims=True)

    stat_spec = lambda k: pl.BlockSpec((1, k), lambda i: (0, 0))
    return pl.pallas_call(
        body,
        grid=(_NB,),
        in_specs=[
            pl.BlockSpec((_BB, AL), lambda i: (i, 0)),
            pl.BlockSpec((_BB, ML), lambda i: (i, 0)),
            pl.BlockSpec((_BB, SL), lambda i: (i, 0)),
        ],
        out_specs=[stat_spec(AL), stat_spec(AL), stat_spec(ML),
                   stat_spec(ML), stat_spec(SL), stat_spec(SL)],
        out_shape=[jax.ShapeDtypeStruct((1, k), jnp.float32)
                   for k in (AL, AL, ML, ML, SL, SL)],
    )(art, mom, feat)


def _dense(uemb, memb, age_f, gen_f, sing_f, genr_f,
           age_t, gender_t, singer_t, genre_t, art, mom, feat,
           w_uf, b_uf, w_ml, b_ml, w_sf, b_sf,
           g_art, be_art, g_mom, be_mom, g_feat, be_feat,
           sa, qa, sm, qm, sf, qf):
    def one_hot(col_ref, n):
        iota = lax.broadcasted_iota(jnp.int32, (_BB, n), 1).astype(jnp.float32)
        return jnp.where(col_ref[...] == iota, 1.0, 0.0)

    def body(ue_ref, me_ref, ag_ref, u_ref, si_ref, gr_ref,
             at_ref, gt_ref, st_ref, grt_ref, a_ref, m_ref, f_ref,
             wa, ba, wm, bm, wf, bf,
             ga, bea, gm, bem, gf, bef,
             sa_r, qa_r, sm_r, qm_r, sf_r, qf_r, out_ref):
        out_ref[:, 0:H] = ue_ref[...]
        out_ref[:, H:2 * H] = jnp.dot(one_hot(ag_ref, 6), at_ref[...],
                                      preferred_element_type=jnp.float32)
        g0 = gt_ref[0:1, :]
        out_ref[:, 2 * H:3 * H] = g0 + u_ref[...] * (gt_ref[1:2, :] - g0)
        for x_ref, w_ref, b_ref, g_ref, be_ref, s_ref, q_ref, off in (
                (a_ref, wa, ba, ga, bea, sa_r, qa_r, 3 * H),
                (m_ref, wm, bm, gm, bem, sm_r, qm_r, 4 * H),
                (f_ref, wf, bf, gf, bef, sf_r, qf_r, 5 * H)):
            mu = s_ref[...] * (1.0 / B)
            var = q_ref[...] * (1.0 / B) - mu * mu
            sc = g_ref[...] / jnp.sqrt(var + _EPS)
            sh = be_ref[...] - mu * sc
            xn = x_ref[...] * sc + sh
            y = jnp.dot(xn, w_ref[...], preferred_element_type=jnp.float32)
            out_ref[:, off:off + H] = y + b_ref[...]
        out_ref[:, 6 * H:7 * H] = jnp.dot(one_hot(si_ref, 417), st_ref[...],
                                          preferred_element_type=jnp.float32)
        out_ref[:, 7 * H:8 * H] = jnp.dot(one_hot(gr_ref, 18), grt_ref[...],
                                          preferred_element_type=jnp.float32)
        out_ref[:, 8 * H:9 * H] = me_ref[...]

    full = lambda r, c: pl.BlockSpec((r, c), lambda i: (0, 0))
    row64 = pl.BlockSpec((_BB, H), lambda i: (i, 0))
    col1 = pl.BlockSpec((_BB, 1), lambda i: (i, 0))
    return pl.pallas_call(
        body,
        grid=(_NB,),
        in_specs=[
            row64, row64, col1, col1, col1, col1,
            full(6, H), full(2, H), full(417, H), full(18, H),
            pl.BlockSpec((_BB, AL), lambda i: (i, 0)),
            pl.BlockSpec((_BB, ML), lambda i: (i, 0)),
            pl.BlockSpec((_BB, SL), lambda i: (i, 0)),
            full(AL, H), full(1, H), full(ML, H), full(1, H),
            full(SL, H), full(1, H),
            full(1, AL), full(1, AL), full(1, ML), full(1, ML),
            full(1, SL), full(1, SL),
            full(1, AL), full(1, AL), full(1, ML), full(1, ML),
            full(1, SL), full(1, SL),
        ],
        out_specs=pl.BlockSpec((_BB, OUT_COLS), lambda i: (i, 0)),
        out_shape=jax.ShapeDtypeStruct((B, OUT_COLS), jnp.float32),
    )(uemb, memb, age_f, gen_f, sing_f, genr_f,
      age_t, gender_t, singer_t, genre_t, art, mom, feat,
      w_uf, b_uf.reshape(1, H), w_ml, b_ml.reshape(1, H),
      w_sf, b_sf.reshape(1, H),
      g_art.reshape(1, AL), be_art.reshape(1, AL),
      g_mom.reshape(1, ML), be_mom.reshape(1, ML),
      g_feat.reshape(1, SL), be_feat.reshape(1, SL),
      sa, qa, sm, qm, sf, qf)


def kernel(user_id, user_age, user_gender, user_articles, user_moments,
           music_id, music_singer, music_genre, music_features,
           UserEmb, AgeEmb, GenderEmb, SingerEmb, GenreEmb, MusicEmb,
           W_uf, b_uf, W_ml, b_ml, W_sf, b_sf,
           g_art, beta_art, g_mom, beta_mom, g_feat, beta_feat):
    uemb, memb = _sc_gather(user_id, music_id, UserEmb, MusicEmb)
    sa, qa, sm, qm, sf, qf = _stats(user_articles, user_moments,
                                    music_features)
    colf = lambda x: x.astype(jnp.float32).reshape(B, 1)
    return _dense(uemb, memb, colf(user_age), colf(user_gender),
                  colf(music_singer), colf(music_genre),
                  AgeEmb, GenderEmb, SingerEmb, GenreEmb,
                  user_articles, user_moments, music_features,
                  W_uf, b_uf, W_ml, b_ml, W_sf, b_sf,
                  g_art, beta_art, g_mom, beta_mom, g_feat, beta_feat,
                  sa, qa, sm, qm, sf, qf)
